# Initial kernel scaffold; baseline (speedup 1.0000x reference)
import jax
import jax.numpy as jnp
from jax.experimental import pallas as pl


def kernel(q, k, v, Wq, bq, Wk, bk, Wv, bv, top_k):
    raise NotImplementedError("write your pallas kernel here")


# speedup vs baseline: 14.0557x; 14.0557x over previous
import functools

import jax
import jax.numpy as jnp
from jax import lax
from jax.experimental import pallas as pl
from jax.experimental.pallas import tpu as pltpu
from jax.experimental.pallas import tpu_sc as plsc

KTOP = 16
QB = 256
KB = 512
CHUNK = 128

_DOT11 = (((1,), (1,)), ((), ()))


def _proj_body(k_ref, v_ref, wk_ref, bk_ref, wv_ref, bv_ref, kp_ref, vp_ref):
    kb = k_ref[0]
    vb = v_ref[0]
    kp_ref[0] = lax.dot_general(kb, wk_ref[...], _DOT11,
                                preferred_element_type=jnp.float32) + bk_ref[...]
    vp_ref[0] = lax.dot_general(vb, wv_ref[...], _DOT11,
                                preferred_element_type=jnp.float32) + bv_ref[...]


def _select_body(q_ref, k_ref, wq_ref, bq_ref, qe_ref, idx_ref, *, nk):
    b = pl.program_id(0)
    qb = q_ref[0]
    kb = k_ref[0]
    qe_ref[0] = lax.dot_general(qb, wq_ref[...], _DOT11,
                                preferred_element_type=jnp.float32) + bq_ref[...]
    q2 = jnp.sum(qb * qb, axis=1)
    k2 = jnp.sum(kb * kb, axis=1)
    qk = lax.dot_general(qb, kb, _DOT11, preferred_element_type=jnp.float32)
    s = q2[:, None] + k2[None, :] - 2.0 * qk
    iota = lax.broadcasted_iota(jnp.int32, (QB, nk), 1)
    big = jnp.float32(3.0e38)
    cols = []
    for _ in range(KTOP):
        m = jnp.min(s, axis=1, keepdims=True)
        idxv = jnp.min(jnp.where(s == m, iota, nk), axis=1)
        cols.append(idxv)
        s = jnp.where(iota == idxv[:, None], big, s)
    idx_ref[0] = jnp.stack(cols, axis=1) + b * nk


def _attn_body(qe_ref, kg_ref, vg_ref, o_ref):
    qe = qe_ref[0]
    svals = [jnp.sum(qe * kg_ref[0, :, i, :], axis=1) for i in range(KTOP)]
    s = jnp.stack(svals, axis=1)
    m = jnp.max(s, axis=1, keepdims=True)
    e = jnp.exp(s - m)
    a = e / jnp.sum(e, axis=1, keepdims=True)
    acc = a[:, 0][:, None] * vg_ref[0, :, 0, :]
    for i in range(1, KTOP):
        acc = acc + a[:, i][:, None] * vg_ref[0, :, i, :]
    o_ref[0] = acc


def _gather_pairs(kp2, vp2, idxf):
    n, h = idxf.shape[0], kp2.shape[1]
    info = plsc.get_sparse_core_info()
    nw = info.num_cores * info.num_subcores
    per_w = n // nw
    nchunk = per_w // CHUNK
    mesh = plsc.VectorSubcoreMesh(core_axis_name="c", subcore_axis_name="s")

    @functools.partial(
        pl.kernel, mesh=mesh,
        out_type=(jax.ShapeDtypeStruct((n, h), jnp.float32),
                  jax.ShapeDtypeStruct((n, h), jnp.float32)),
        scratch_types=[pltpu.VMEM((CHUNK,), jnp.int32),
                       pltpu.VMEM((CHUNK, h), jnp.float32),
                       pltpu.VMEM((CHUNK, h), jnp.float32),
                       pltpu.SemaphoreType.DMA,
                       pltpu.SemaphoreType.DMA],
    )
    def body(kp_hbm, vp_hbm, idx_hbm, kg_hbm, vg_hbm, idx_v, kbuf, vbuf, sk, sv):
        wid = lax.axis_index("s") * info.num_cores + lax.axis_index("c")
        base = wid * per_w

        def step(c, carry):
            off = base + c * CHUNK
            pltpu.sync_copy(idx_hbm.at[pl.ds(off, CHUNK)], idx_v)
            ck = pltpu.async_copy(kp_hbm.at[idx_v], kbuf, sk)
            cv = pltpu.async_copy(vp_hbm.at[idx_v], vbuf, sv)
            ck.wait()
            pltpu.sync_copy(kbuf, kg_hbm.at[pl.ds(off, CHUNK)])
            cv.wait()
            pltpu.sync_copy(vbuf, vg_hbm.at[pl.ds(off, CHUNK)])
            return carry

        lax.fori_loop(0, nchunk, step, 0)

    return body(kp2, vp2, idxf)


def kernel(q, k, v, Wq, bq, Wk, bk, Wv, bv, top_k):
    b, nq, c = q.shape
    nk = k.shape[1]
    h = Wq.shape[0]
    bq2, bk2, bv2 = (x.reshape(1, h) for x in (bq, bk, bv))
    f32 = jnp.float32

    w_spec = pl.BlockSpec((h, c), lambda i, j: (0, 0))
    b_spec = pl.BlockSpec((1, h), lambda i, j: (0, 0))

    kp, vp = pl.pallas_call(
        _proj_body,
        grid=(b, nk // KB),
        in_specs=[pl.BlockSpec((1, KB, c), lambda i, j: (i, j, 0)),
                  pl.BlockSpec((1, KB, c), lambda i, j: (i, j, 0)),
                  w_spec, b_spec, w_spec, b_spec],
        out_specs=[pl.BlockSpec((1, KB, h), lambda i, j: (i, j, 0)),
                   pl.BlockSpec((1, KB, h), lambda i, j: (i, j, 0))],
        out_shape=[jax.ShapeDtypeStruct((b, nk, h), f32),
                   jax.ShapeDtypeStruct((b, nk, h), f32)],
    )(k, v, Wk, bk2, Wv, bv2)

    qe, idx = pl.pallas_call(
        functools.partial(_select_body, nk=nk),
        grid=(b, nq // QB),
        in_specs=[pl.BlockSpec((1, QB, c), lambda i, j: (i, j, 0)),
                  pl.BlockSpec((1, nk, c), lambda i, j: (i, 0, 0)),
                  w_spec, b_spec],
        out_specs=[pl.BlockSpec((1, QB, h), lambda i, j: (i, j, 0)),
                   pl.BlockSpec((1, QB, KTOP), lambda i, j: (i, j, 0))],
        out_shape=[jax.ShapeDtypeStruct((b, nq, h), f32),
                   jax.ShapeDtypeStruct((b, nq, KTOP), jnp.int32)],
    )(q, k, Wq, bq2)

    kg, vg = _gather_pairs(kp.reshape(b * nk, h), vp.reshape(b * nk, h),
                           idx.reshape(-1))

    out = pl.pallas_call(
        _attn_body,
        grid=(b, nq // QB),
        in_specs=[pl.BlockSpec((1, QB, h), lambda i, j: (i, j, 0)),
                  pl.BlockSpec((1, QB, KTOP, h), lambda i, j: (i, j, 0, 0)),
                  pl.BlockSpec((1, QB, KTOP, h), lambda i, j: (i, j, 0, 0))],
        out_specs=pl.BlockSpec((1, QB, h), lambda i, j: (i, j, 0)),
        out_shape=jax.ShapeDtypeStruct((b, nq, h), f32),
    )(qe, kg.reshape(b, nq, KTOP, h), vg.reshape(b, nq, KTOP, h))

    return out


# speedup vs baseline: 18.7784x; 1.3360x over previous
import functools

import jax
import jax.numpy as jnp
from jax import lax
from jax.experimental import pallas as pl
from jax.experimental.pallas import tpu as pltpu
from jax.experimental.pallas import tpu_sc as plsc

KTOP = 16
QB = 256
KB = 512
CHUNK = 128

_DOT11 = (((1,), (1,)), ((), ()))


def _proj_body(k_ref, v_ref, wk_ref, bk_ref, wv_ref, bv_ref, kp_ref, vp_ref):
    kb = k_ref[0]
    vb = v_ref[0]
    kp_ref[0] = lax.dot_general(kb, wk_ref[...], _DOT11,
                                preferred_element_type=jnp.float32) + bk_ref[...]
    vp_ref[0] = lax.dot_general(vb, wv_ref[...], _DOT11,
                                preferred_element_type=jnp.float32) + bv_ref[...]


def _select_body(q_ref, k_ref, wq_ref, bq_ref, qe_ref, idx_ref, *, nk, boff):
    qb = q_ref[...]
    kb = k_ref[...]
    qe_ref[...] = lax.dot_general(qb, wq_ref[...], _DOT11,
                                  preferred_element_type=jnp.float32) + bq_ref[...]
    q2 = jnp.sum(qb * qb, axis=1)
    k2 = jnp.sum(kb * kb, axis=1)
    qk = lax.dot_general(qb, kb, _DOT11, preferred_element_type=jnp.float32)
    s = q2[:, None] + k2[None, :] - 2.0 * qk
    fiota = lax.broadcasted_iota(jnp.int32, (QB, nk), 1).astype(jnp.float32)
    big = jnp.float32(3.0e38)
    cols = []
    for _ in range(KTOP):
        m = jnp.min(s, axis=1, keepdims=True)
        fidx = jnp.min(jnp.where(s == m, fiota, big), axis=1)
        cols.append(fidx)
        s = jnp.where(fiota == fidx[:, None], big, s)
    idx_ref[...] = jnp.stack(cols, axis=1).astype(jnp.int32) + boff


def _attn_body(qe_ref, kg_ref, vg_ref, o_ref):
    qe = qe_ref[...]
    s = jnp.sum(qe[:, None, :] * kg_ref[...], axis=2)
    m = jnp.max(s, axis=1, keepdims=True)
    e = jnp.exp(s - m)
    a = e / jnp.sum(e, axis=1, keepdims=True)
    o_ref[...] = jnp.sum(a[:, :, None] * vg_ref[...], axis=1)


def _gather_pairs(kp2, vp2, idxf):
    n, h = idxf.shape[0], kp2.shape[1]
    info = plsc.get_sparse_core_info()
    nw = info.num_cores * info.num_subcores
    per_w = n // nw
    nchunk = per_w // CHUNK
    mesh = plsc.VectorSubcoreMesh(core_axis_name="c", subcore_axis_name="s")

    @functools.partial(
        pl.kernel, mesh=mesh,
        out_type=(jax.ShapeDtypeStruct((n, h), jnp.float32),
                  jax.ShapeDtypeStruct((n, h), jnp.float32)),
        scratch_types=[pltpu.VMEM((CHUNK,), jnp.int32),
                       pltpu.VMEM((CHUNK, h), jnp.float32),
                       pltpu.VMEM((CHUNK, h), jnp.float32),
                       pltpu.SemaphoreType.DMA,
                       pltpu.SemaphoreType.DMA],
    )
    def body(kp_hbm, vp_hbm, idx_hbm, kg_hbm, vg_hbm, idx_v, kbuf, vbuf, sk, sv):
        wid = lax.axis_index("s") * info.num_cores + lax.axis_index("c")
        base = wid * per_w

        def step(c, carry):
            off = base + c * CHUNK
            pltpu.sync_copy(idx_hbm.at[pl.ds(off, CHUNK)], idx_v)
            ck = pltpu.async_copy(kp_hbm.at[idx_v], kbuf, sk)
            cv = pltpu.async_copy(vp_hbm.at[idx_v], vbuf, sv)
            ck.wait()
            pltpu.sync_copy(kbuf, kg_hbm.at[pl.ds(off, CHUNK)])
            cv.wait()
            pltpu.sync_copy(vbuf, vg_hbm.at[pl.ds(off, CHUNK)])
            return carry

        lax.fori_loop(0, nchunk, step, 0)

    return body(kp2, vp2, idxf)


def kernel(q, k, v, Wq, bq, Wk, bk, Wv, bv, top_k):
    b, nq, c = q.shape
    nk = k.shape[1]
    h = Wq.shape[0]
    bq2, bk2, bv2 = (x.reshape(1, h) for x in (bq, bk, bv))
    f32 = jnp.float32

    w2_spec = pl.BlockSpec((h, c), lambda i, j: (0, 0))
    b2_spec = pl.BlockSpec((1, h), lambda i, j: (0, 0))
    w_spec = pl.BlockSpec((h, c), lambda i: (0, 0))
    b_spec = pl.BlockSpec((1, h), lambda i: (0, 0))

    kp, vp = pl.pallas_call(
        _proj_body,
        grid=(b, nk // KB),
        in_specs=[pl.BlockSpec((1, KB, c), lambda i, j: (i, j, 0)),
                  pl.BlockSpec((1, KB, c), lambda i, j: (i, j, 0)),
                  w2_spec, b2_spec, w2_spec, b2_spec],
        out_specs=[pl.BlockSpec((1, KB, h), lambda i, j: (i, j, 0)),
                   pl.BlockSpec((1, KB, h), lambda i, j: (i, j, 0))],
        out_shape=[jax.ShapeDtypeStruct((b, nk, h), f32),
                   jax.ShapeDtypeStruct((b, nk, h), f32)],
    )(k, v, Wk, bk2, Wv, bv2)
    kp2 = kp.reshape(b * nk, h)
    vp2 = vp.reshape(b * nk, h)

    outs = []
    for bi in range(b):
        qe, idx = pl.pallas_call(
            functools.partial(_select_body, nk=nk, boff=bi * nk),
            grid=(nq // QB,),
            in_specs=[pl.BlockSpec((QB, c), lambda i: (i, 0)),
                      pl.BlockSpec((nk, c), lambda i: (0, 0)),
                      w_spec, b_spec],
            out_specs=[pl.BlockSpec((QB, h), lambda i: (i, 0)),
                       pl.BlockSpec((QB, KTOP), lambda i: (i, 0))],
            out_shape=[jax.ShapeDtypeStruct((nq, h), f32),
                       jax.ShapeDtypeStruct((nq, KTOP), jnp.int32)],
        )(q[bi], k[bi], Wq, bq2)

        kg, vg = _gather_pairs(kp2, vp2, idx.reshape(-1))

        out_b = pl.pallas_call(
            _attn_body,
            grid=(nq // QB,),
            in_specs=[pl.BlockSpec((QB, h), lambda i: (i, 0)),
                      pl.BlockSpec((QB, KTOP, h), lambda i: (i, 0, 0)),
                      pl.BlockSpec((QB, KTOP, h), lambda i: (i, 0, 0))],
            out_specs=pl.BlockSpec((QB, h), lambda i: (i, 0)),
            out_shape=jax.ShapeDtypeStruct((nq, h), f32),
        )(qe, kg.reshape(nq, KTOP, h), vg.reshape(nq, KTOP, h))
        outs.append(out_b)

    return jnp.stack(outs, axis=0)
